# baseline (speedup 1.0000x reference)
import jax
import jax.numpy as jnp
from jax import lax
from jax.experimental import pallas as pl
from jax.experimental.pallas import tpu as pltpu
from jax.experimental.pallas import tpu_sc as plsc

VOCAB = 100000
EMBED = 64
BATCH = 16384
LANES = 16
NC = 2
NS = 16
NW = NC * NS
BPW = BATCH // NW
CHUNK = 128
HALF = BPW // 2
GROUPS = HALF // LANES


def _glove_body(wu_hbm, wv_hbm, wuh_hbm, wvh_hbm, in_embed_hbm, in_bias_hbm,
                out_embed_hbm, out_bias_hbm, out_hbm, idx_u, idx_v, idxh_u,
                idxh_v, u_rows, v_rows, u_bias, v_bias, out_buf, sem):
    wid = lax.axis_index("s") * NC + lax.axis_index("c")
    base = wid * BPW

    pltpu.sync_copy(wu_hbm.at[pl.ds(base, BPW)], idx_u)
    pltpu.sync_copy(wv_hbm.at[pl.ds(base, BPW)], idx_v)
    pltpu.sync_copy(wuh_hbm.at[pl.ds(base, BPW)], idxh_u)
    pltpu.sync_copy(wvh_hbm.at[pl.ds(base, BPW)], idxh_v)

    bias_copies = []
    for j in range(BPW // CHUNK):
        sl = pl.ds(j * CHUNK, CHUNK)
        bias_copies.append(pltpu.make_async_copy(
            in_bias_hbm.at[idx_u.at[sl]], u_bias.at[sl], sem))
        bias_copies.append(pltpu.make_async_copy(
            out_bias_hbm.at[idx_v.at[sl]], v_bias.at[sl], sem))
    for c in bias_copies:
        c.start()

    lane = lax.iota(jnp.int32, LANES)

    for p in range(2):
        row_copies = []
        for j in range(HALF // CHUNK):
            isl = pl.ds(p * HALF + j * CHUNK, CHUNK)
            dsl = pl.ds(j * CHUNK, CHUNK)
            row_copies.append(pltpu.make_async_copy(
                in_embed_hbm.at[idxh_u.at[isl]], u_rows.at[dsl], sem))
            row_copies.append(pltpu.make_async_copy(
                out_embed_hbm.at[idxh_v.at[isl]], v_rows.at[dsl], sem))
        for c in row_copies:
            c.start()
        if p == 0:
            for c in bias_copies:
                c.wait()
        for c in row_copies:
            c.wait()

        def group(g, carry):
            rbase = g * LANES
            pu = idx_u[pl.ds(p * HALF + rbase, LANES)] & 1
            pv = idx_v[pl.ds(p * HALF + rbase, LANES)] & 1
            acc = jnp.zeros((LANES,), jnp.float32)
            for r in range(LANES):
                row = rbase + r
                pur = pu[r] != 0
                pvr = pv[r] != 0
                s = jnp.zeros((LANES,), jnp.float32)
                for c in range(4):
                    uc = jnp.where(pur, u_rows[row, pl.ds(64 + c * 16, 16)],
                                   u_rows[row, pl.ds(c * 16, 16)])
                    vc = jnp.where(pvr, v_rows[row, pl.ds(64 + c * 16, 16)],
                                   v_rows[row, pl.ds(c * 16, 16)])
                    s = s + uc * vc
                total = jnp.sum(s, axis=0)
                acc = jnp.where(lane == r, total, acc)
            sl16 = pl.ds(p * HALF + rbase, LANES)
            out_buf[sl16] = acc + u_bias[sl16] + v_bias[sl16]
            return carry

        lax.fori_loop(0, GROUPS, group, 0)

    pltpu.sync_copy(out_buf, out_hbm.at[pl.ds(base, BPW)])


def _glove_sc(wu, wv, wuh, wvh, embed_u2, bias_u1, embed_v2, bias_v1):
    mesh = plsc.VectorSubcoreMesh(core_axis_name="c", subcore_axis_name="s")
    f = pl.kernel(
        _glove_body,
        out_type=jax.ShapeDtypeStruct((BATCH,), jnp.float32),
        mesh=mesh,
        scratch_types=[
            pltpu.VMEM((BPW,), jnp.int32),
            pltpu.VMEM((BPW,), jnp.int32),
            pltpu.VMEM((BPW,), jnp.int32),
            pltpu.VMEM((BPW,), jnp.int32),
            pltpu.VMEM((HALF, 2 * EMBED), jnp.float32),
            pltpu.VMEM((HALF, 2 * EMBED), jnp.float32),
            pltpu.VMEM((BPW,), jnp.float32),
            pltpu.VMEM((BPW,), jnp.float32),
            pltpu.VMEM((BPW,), jnp.float32),
            pltpu.SemaphoreType.DMA,
        ],
        compiler_params=pltpu.CompilerParams(needs_layout_passes=False),
    )
    return f(wu, wv, wuh, wvh, embed_u2, bias_u1, embed_v2, bias_v1)


def kernel(word_u, word_v, in_embed, in_bias, out_embed, out_bias):
    wu = word_u.astype(jnp.int32)
    wv = word_v.astype(jnp.int32)
    return _glove_sc(wu, wv, wu >> 1, wv >> 1,
                     in_embed.reshape(VOCAB // 2, 2 * EMBED),
                     in_bias.reshape(VOCAB),
                     out_embed.reshape(VOCAB // 2, 2 * EMBED),
                     out_bias.reshape(VOCAB))


# speedup vs baseline: 1.7293x; 1.7293x over previous
import jax
import jax.numpy as jnp
from jax import lax
from jax.experimental import pallas as pl
from jax.experimental.pallas import tpu as pltpu
from jax.experimental.pallas import tpu_sc as plsc

VOCAB = 100000
EMBED = 64
BATCH = 16384
LANES = 16
NC = 2
NS = 16
W = 128
ROWS = BATCH // W
RPW = ROWS // NS
ICH = 4096
NCH = BATCH // ICH
CROWS = ICH // W
PAIRS_PER_W = EMBED // (NC * NS)


def _glove_body(wu_hbm, wv_hbm, ut_hbm, ub_hbm, vt_hbm, vb_hbm,
                out0_hbm, out1_hbm, vec, g1, idxb, sbuf, bbuf, bidx, ridx,
                shared, sem):
    c = lax.axis_index("c")
    s = lax.axis_index("s")
    lane = lax.iota(jnp.int32, LANES)
    zero16 = jnp.zeros((LANES,), jnp.float32)

    for i in range(W // LANES):
        ridx[pl.ds(i * LANES, LANES)] = lane + i * LANES

    for r in range(RPW):
        for q in range(W // LANES):
            sbuf[r, pl.ds(q * LANES, LANES)] = zero16
    my_rows = pl.ds(s * RPW, RPW)
    pltpu.sync_copy(sbuf, shared.at[my_rows])
    plsc.subcore_barrier()

    for k in range(PAIRS_PER_W):
        f = c * (NS * PAIRS_PER_W) + s * PAIRS_PER_W + k

        pltpu.sync_copy(ut_hbm.at[f], vec)
        for ch in range(NCH):
            pltpu.sync_copy(wu_hbm.at[pl.ds(ch * ICH, ICH)], idxb)

            def gu(r2, carry, _ch=ch):
                row = _ch * CROWS + r2
                for q in range(W // LANES):
                    g1[row, pl.ds(q * LANES, LANES)] = plsc.load_gather(
                        vec, [idxb[pl.ds(r2 * W + q * LANES, LANES)]])
                return carry
            lax.fori_loop(0, CROWS, gu, 0)

        pltpu.sync_copy(vt_hbm.at[f], vec)
        for ch in range(NCH):
            pltpu.sync_copy(wv_hbm.at[pl.ds(ch * ICH, ICH)], idxb)

            def gv(r2, carry, _ch=ch):
                row = _ch * CROWS + r2
                for q in range(W // LANES):
                    sl = pl.ds(q * LANES, LANES)
                    g1[row, sl] = g1[row, sl] * plsc.load_gather(
                        vec, [idxb[pl.ds(r2 * W + q * LANES, LANES)]])
                return carry
            lax.fori_loop(0, CROWS, gv, 0)

        pltpu.sync_copy(g1, shared.at[ridx], add=True)

    plsc.subcore_barrier()

    pltpu.sync_copy(shared.at[my_rows], sbuf)

    @pl.when(c == 0)
    def _():
        for bias_hbm, widx_hbm in ((ub_hbm, wu_hbm), (vb_hbm, wv_hbm)):
            pltpu.sync_copy(widx_hbm.at[pl.ds(s * RPW * W, RPW * W)], bidx)
            copies = [pltpu.make_async_copy(
                bias_hbm.at[bidx.at[pl.ds(j * W, W)]],
                bbuf.at[pl.ds(j * W, W)], sem)
                for j in range(RPW)]
            for cp in copies:
                cp.start()
            for cp in copies:
                cp.wait()
            for r in range(RPW):
                for q in range(W // LANES):
                    sl = pl.ds(q * LANES, LANES)
                    sbuf[r, sl] = sbuf[r, sl] + bbuf[pl.ds(r * W + q * LANES,
                                                           LANES)]
        pltpu.sync_copy(sbuf, out0_hbm.at[my_rows])

    @pl.when(c == 1)
    def _():
        pltpu.sync_copy(sbuf, out1_hbm.at[my_rows])


def _glove_sc(wu, wv, ut, ub1, vt, vb1):
    mesh = plsc.VectorSubcoreMesh(core_axis_name="c", subcore_axis_name="s")
    f = pl.kernel(
        _glove_body,
        out_type=(jax.ShapeDtypeStruct((ROWS, W), jnp.float32),
                  jax.ShapeDtypeStruct((ROWS, W), jnp.float32)),
        mesh=mesh,
        scratch_types=[
            pltpu.VMEM((VOCAB,), jnp.float32),
            pltpu.VMEM((ROWS, W), jnp.float32),
            pltpu.VMEM((ICH,), jnp.int32),
            pltpu.VMEM((RPW, W), jnp.float32),
            pltpu.VMEM((RPW * W,), jnp.float32),
            pltpu.VMEM((RPW * W,), jnp.int32),
            pltpu.VMEM((W,), jnp.int32),
            pltpu.VMEM_SHARED((ROWS, W), jnp.float32),
            pltpu.SemaphoreType.DMA,
        ],
        compiler_params=pltpu.CompilerParams(needs_layout_passes=False),
    )
    return f(wu, wv, ut, ub1, vt, vb1)


def kernel(word_u, word_v, in_embed, in_bias, out_embed, out_bias):
    wu = word_u.astype(jnp.int32)
    wv = word_v.astype(jnp.int32)
    out0, out1 = _glove_sc(wu, wv, in_embed.T, in_bias.reshape(VOCAB),
                           out_embed.T, out_bias.reshape(VOCAB))
    return (out0 + out1).reshape(BATCH)


# speedup vs baseline: 1.9151x; 1.1074x over previous
import jax
import jax.numpy as jnp
from jax import lax
from jax.experimental import pallas as pl
from jax.experimental.pallas import tpu as pltpu
from jax.experimental.pallas import tpu_sc as plsc

VOCAB = 100000
EMBED = 64
BATCH = 16384
LANES = 16
NC = 2
NS = 16
W = 128
ROWS = BATCH // W
RPW = ROWS // NS
ICH = 4096
NCH = BATCH // ICH
CROWS = ICH // W
PAIRS_PER_W = EMBED // (NC * NS)


def _glove_body(wu_hbm, wv_hbm, ut_hbm, ub_hbm, vt_hbm, vb_hbm,
                out0_hbm, out1_hbm, vec, g1, idxb, sbuf, bbuf, bidx, ridx,
                shared, sem):
    c = lax.axis_index("c")
    s = lax.axis_index("s")
    lane = lax.iota(jnp.int32, LANES)
    zero16 = jnp.zeros((LANES,), jnp.float32)

    for i in range(W // LANES):
        ridx[pl.ds(i * LANES, LANES)] = lane + i * LANES

    for r in range(RPW):
        for q in range(W // LANES):
            sbuf[r, pl.ds(q * LANES, LANES)] = zero16
    my_rows = pl.ds(s * RPW, RPW)
    pltpu.sync_copy(sbuf, shared.at[my_rows])
    plsc.subcore_barrier()

    for k in range(PAIRS_PER_W):
        f = c * (NS * PAIRS_PER_W) + s * PAIRS_PER_W + k

        pltpu.sync_copy(ut_hbm.at[f], vec)
        for ch in range(NCH):
            pltpu.sync_copy(wu_hbm.at[pl.ds(ch * ICH, ICH)], idxb)

            @plsc.parallel_loop(0, CROWS, unroll=4)
            def gu(r2, _ch=ch):
                row = _ch * CROWS + r2
                for q in range(W // LANES):
                    g1[row, pl.ds(q * LANES, LANES)] = plsc.load_gather(
                        vec, [idxb[pl.ds(r2 * W + q * LANES, LANES)]])

        pltpu.sync_copy(vt_hbm.at[f], vec)
        for ch in range(NCH):
            pltpu.sync_copy(wv_hbm.at[pl.ds(ch * ICH, ICH)], idxb)

            @plsc.parallel_loop(0, CROWS, unroll=4)
            def gv(r2, _ch=ch):
                row = _ch * CROWS + r2
                for q in range(W // LANES):
                    sl = pl.ds(q * LANES, LANES)
                    g1[row, sl] = g1[row, sl] * plsc.load_gather(
                        vec, [idxb[pl.ds(r2 * W + q * LANES, LANES)]])

        pltpu.sync_copy(g1, shared.at[ridx], add=True)

    plsc.subcore_barrier()

    pltpu.sync_copy(shared.at[my_rows], sbuf)

    @pl.when(c == 0)
    def _():
        for bias_hbm, widx_hbm in ((ub_hbm, wu_hbm), (vb_hbm, wv_hbm)):
            pltpu.sync_copy(widx_hbm.at[pl.ds(s * RPW * W, RPW * W)], bidx)
            copies = [pltpu.make_async_copy(
                bias_hbm.at[bidx.at[pl.ds(j * W, W)]],
                bbuf.at[pl.ds(j * W, W)], sem)
                for j in range(RPW)]
            for cp in copies:
                cp.start()
            for cp in copies:
                cp.wait()
            for r in range(RPW):
                for q in range(W // LANES):
                    sl = pl.ds(q * LANES, LANES)
                    sbuf[r, sl] = sbuf[r, sl] + bbuf[pl.ds(r * W + q * LANES,
                                                           LANES)]
        pltpu.sync_copy(sbuf, out0_hbm.at[my_rows])

    @pl.when(c == 1)
    def _():
        pltpu.sync_copy(sbuf, out1_hbm.at[my_rows])


def _glove_sc(wu, wv, ut, ub1, vt, vb1):
    mesh = plsc.VectorSubcoreMesh(core_axis_name="c", subcore_axis_name="s")
    f = pl.kernel(
        _glove_body,
        out_type=(jax.ShapeDtypeStruct((ROWS, W), jnp.float32),
                  jax.ShapeDtypeStruct((ROWS, W), jnp.float32)),
        mesh=mesh,
        scratch_types=[
            pltpu.VMEM((VOCAB,), jnp.float32),
            pltpu.VMEM((ROWS, W), jnp.float32),
            pltpu.VMEM((ICH,), jnp.int32),
            pltpu.VMEM((RPW, W), jnp.float32),
            pltpu.VMEM((RPW * W,), jnp.float32),
            pltpu.VMEM((RPW * W,), jnp.int32),
            pltpu.VMEM((W,), jnp.int32),
            pltpu.VMEM_SHARED((ROWS, W), jnp.float32),
            pltpu.SemaphoreType.DMA,
        ],
        compiler_params=pltpu.CompilerParams(needs_layout_passes=False),
    )
    return f(wu, wv, ut, ub1, vt, vb1)


def kernel(word_u, word_v, in_embed, in_bias, out_embed, out_bias):
    wu = word_u.astype(jnp.int32)
    wv = word_v.astype(jnp.int32)
    out0, out1 = _glove_sc(wu, wv, in_embed.T, in_bias.reshape(VOCAB),
                           out_embed.T, out_bias.reshape(VOCAB))
    return (out0 + out1).reshape(BATCH)


# speedup vs baseline: 2.0413x; 1.0659x over previous
import jax
import jax.numpy as jnp
from jax import lax
from jax.experimental import pallas as pl
from jax.experimental.pallas import tpu as pltpu
from jax.experimental.pallas import tpu_sc as plsc

VOCAB = 100000
EMBED = 64
BATCH = 16384
LANES = 16
NC = 2
NS = 16
W = 128
ROWS = BATCH // W
RPW = ROWS // NS
VH = 49920
TAIL0 = 2 * VH
TAIL = VOCAB - TAIL0
ICH = 8192
IROWS = ICH // W
PAIRS_PER_W = EMBED // (NC * NS)
NSTREAM = PAIRS_PER_W * 2 * 2


def _glove_body(wu_hbm, wv_hbm, ut_hbm, ub_hbm, vt_hbm, vb_hbm,
                tu_hbm, tv_hbm, out0_hbm, out1_hbm, vec0, vec1, g1, idxb, tailb,
                sbuf, bbuf, bidx, ridx, shared, sidx_u, sidx_v, sem, sem2):
    c = lax.axis_index("c")
    s = lax.axis_index("s")
    lane = lax.iota(jnp.int32, LANES)
    zero16 = jnp.zeros((LANES,), jnp.float32)
    fs = [c * (NS * PAIRS_PER_W) + s * PAIRS_PER_W + k
          for k in range(PAIRS_PER_W)]

    for i in range(W // LANES):
        ridx[pl.ds(i * LANES, LANES)] = lane + i * LANES

    @pl.when(s == 0)
    def _():
        pltpu.sync_copy(wu_hbm, sidx_u)
        pltpu.sync_copy(wv_hbm, sidx_v)

    for r in range(RPW):
        for q in range(W // LANES):
            sbuf[r, pl.ds(q * LANES, LANES)] = zero16
    my_rows = pl.ds(s * RPW, RPW)
    pltpu.sync_copy(sbuf, shared.at[my_rows])
    plsc.subcore_barrier()

    def mk(t):
        k, ph, h = t // 4, (t // 2) % 2, t % 2
        tab = vt_hbm if ph else ut_hbm
        return pltpu.make_async_copy(
            tab.at[:, pl.ds(h * VH, VH)].at[fs[k]],
            vec0 if t % 2 == 0 else vec1, sem)

    def scan(ph, h, ch, buf):
        lo = h * VH

        @plsc.parallel_loop(0, IROWS, unroll=1)
        def body(r2):
            r = ch * IROWS + r2
            for qq in range(W // LANES):
                sl = pl.ds(r2 * W + qq * LANES, LANES)
                csl = pl.ds(qq * LANES, LANES)
                iu = idxb[sl]
                li = iu - lo
                m = (li >= 0) & (li < VH)
                g = plsc.load_gather(buf, [jnp.where(m, li, 0)])
                if h == 1:
                    lt = iu - TAIL0
                    mt = lt >= 0
                    gt = plsc.load_gather(tailb, [jnp.where(mt, lt, 0)])
                    g = jnp.where(mt, gt, g)
                    m = m | mt
                if ph == 0:
                    if h == 0:
                        g1[r, csl] = jnp.where(m, g, 0.0)
                    else:
                        g1[r, csl] = g1[r, csl] + jnp.where(m, g, 0.0)
                else:
                    g1[r, csl] = g1[r, csl] * jnp.where(m, g, 1.0)

    mk(0).start()
    mk(1).start()
    for t in range(NSTREAM):
        k, ph, h = t // 4, (t // 2) % 2, t % 2
        if h == 0:
            ttab = tv_hbm if ph else tu_hbm
            pltpu.sync_copy(ttab.at[fs[k]], tailb)
        mk(t).wait()
        idx_sp = sidx_v if ph else sidx_u
        for ch in range(BATCH // ICH):
            pltpu.sync_copy(idx_sp.at[pl.ds(ch * ICH, ICH)], idxb)
            scan(ph, h, ch, vec0 if t % 2 == 0 else vec1)
        if t + 2 < NSTREAM:
            mk(t + 2).start()
        if t % 4 == 3:
            pltpu.sync_copy(g1, shared.at[ridx], add=True)

    plsc.subcore_barrier()

    pltpu.sync_copy(shared.at[my_rows], sbuf)

    def add_bias(bias_hbm, widx_hbm):
        half_rows = RPW // 2
        for hh in range(2):
            base = (s * RPW + hh * half_rows) * W
            pltpu.sync_copy(widx_hbm.at[pl.ds(base, half_rows * W)], bidx)
            copies = [pltpu.make_async_copy(
                bias_hbm.at[bidx.at[pl.ds(j * W, W)]],
                bbuf.at[pl.ds(j * W, W)], sem2)
                for j in range(half_rows)]
            for cp in copies:
                cp.start()
            for cp in copies:
                cp.wait()
            for r in range(half_rows):
                for q in range(W // LANES):
                    sl = pl.ds(q * LANES, LANES)
                    sbuf[hh * half_rows + r, sl] = (
                        sbuf[hh * half_rows + r, sl]
                        + bbuf[pl.ds(r * W + q * LANES, LANES)])

    @pl.when(c == 0)
    def _():
        add_bias(ub_hbm, wu_hbm)
        pltpu.sync_copy(sbuf, out0_hbm.at[my_rows])

    @pl.when(c == 1)
    def _():
        add_bias(vb_hbm, wv_hbm)
        pltpu.sync_copy(sbuf, out1_hbm.at[my_rows])


def _glove_sc(wu, wv, ut, ub1, vt, vb1, tu, tv):
    mesh = plsc.VectorSubcoreMesh(core_axis_name="c", subcore_axis_name="s")
    f = pl.kernel(
        _glove_body,
        out_type=(jax.ShapeDtypeStruct((ROWS, W), jnp.float32),
                  jax.ShapeDtypeStruct((ROWS, W), jnp.float32)),
        mesh=mesh,
        scratch_types=[
            pltpu.VMEM((VH,), jnp.float32),
            pltpu.VMEM((VH,), jnp.float32),
            pltpu.VMEM((ROWS, W), jnp.float32),
            pltpu.VMEM((ICH,), jnp.int32),
            pltpu.VMEM((TAIL,), jnp.float32),
            pltpu.VMEM((RPW, W), jnp.float32),
            pltpu.VMEM((RPW * W // 2,), jnp.float32),
            pltpu.VMEM((RPW * W // 2,), jnp.int32),
            pltpu.VMEM((W,), jnp.int32),
            pltpu.VMEM_SHARED((ROWS, W), jnp.float32),
            pltpu.VMEM_SHARED((BATCH,), jnp.int32),
            pltpu.VMEM_SHARED((BATCH,), jnp.int32),
            pltpu.SemaphoreType.DMA,
            pltpu.SemaphoreType.DMA,
        ],
        compiler_params=pltpu.CompilerParams(needs_layout_passes=False),
    )
    return f(wu, wv, ut, ub1, vt, vb1, tu, tv)


def kernel(word_u, word_v, in_embed, in_bias, out_embed, out_bias):
    wu = word_u.astype(jnp.int32)
    wv = word_v.astype(jnp.int32)
    ut = in_embed.T
    vt = out_embed.T
    out0, out1 = _glove_sc(wu, wv, ut, in_bias.reshape(VOCAB),
                           vt, out_bias.reshape(VOCAB),
                           ut[:, TAIL0:], vt[:, TAIL0:])
    return (out0 + out1).reshape(BATCH)
